# baseline (speedup 1.0000x reference)
import functools

import jax
import jax.numpy as jnp
from jax import lax
from jax.experimental import pallas as pl
from jax.experimental.pallas import tpu as pltpu
from jax.experimental.pallas import tpu_sc as plsc

N = 10000
E = 320000
F = 128
FH = 64
D2 = 48
D2H = 24
DDEG = 16
NCLS = 40

NC, NS = 2, 16
NW = NC * NS
C = 128
NCHUNK = E // C
ZROWS = 632
ACC_ROWS = NS * ZROWS
ROW_TAIL = N - (NS - 1) * ZROWS

BM = 1000
GRID = N // BM


def _rowwise(fn):
    def run(s):
        @pl.when(s < NS - 1)
        def _():
            fn(s * ZROWS, ZROWS)

        @pl.when(s == NS - 1)
        def _():
            fn((NS - 1) * ZROWS, ROW_TAIL)
    return run


def _halfwise(c, fn):
    @pl.when(c == 0)
    def _():
        fn(0)

    @pl.when(c == 1)
    def _():
        fn(1)


@functools.lru_cache(maxsize=None)
def _make_sc_agg(DH, NBUF):
    mesh = plsc.VectorSubcoreMesh(core_axis_name="c", subcore_axis_name="s",
                                  num_cores=NC, num_subcores=NS)
    niter_max = -(-NCHUNK // NS)
    full = NCHUNK - NS * (niter_max - 1)

    @functools.partial(
        pl.kernel,
        out_type=jax.ShapeDtypeStruct((N, 2 * DH), jnp.float32),
        mesh=mesh,
        scratch_types=[
            [pltpu.VMEM((C,), jnp.int32) for _ in range(NBUF)],
            [pltpu.VMEM((C,), jnp.int32) for _ in range(NBUF)],
            [pltpu.VMEM((C, DH), jnp.float32) for _ in range(NBUF)],
            pltpu.VMEM_SHARED((N, DH), jnp.float32),
            pltpu.VMEM_SHARED((ACC_ROWS, DH), jnp.float32),
            [pltpu.SemaphoreType.DMA for _ in range(NBUF)],
            [pltpu.SemaphoreType.DMA for _ in range(NBUF)],
            [pltpu.SemaphoreType.DMA for _ in range(NBUF)],
        ],
        compiler_params=pltpu.CompilerParams(use_tc_tiling_on_sc=False),
    )
    def sc_agg(hw_hbm, eidx_hbm, zeros_hbm, dep_hbm, out_hbm,
               srcb, dstb, rows, tab_sh, acc_sh, sem_s, sem_d, sem_g):
        del dep_hbm
        c = lax.axis_index("c")
        s = lax.axis_index("s")
        niter = jnp.where(s < full, niter_max, niter_max - 1)

        _halfwise(c, lambda h: _rowwise(lambda r0, nr: pltpu.sync_copy(
            hw_hbm.at[pl.ds(r0, nr), pl.ds(h * DH, DH)],
            tab_sh.at[pl.ds(r0, nr)]))(s))
        pltpu.sync_copy(zeros_hbm, acc_sh.at[pl.ds(s * ZROWS, ZROWS)])
        plsc.subcore_barrier()

        def g(i):
            return NS * i + s

        def load_idx(i, p):
            pltpu.async_copy(eidx_hbm.at[0, g(i)], srcb[p], sem_s[p])
            pltpu.async_copy(eidx_hbm.at[1, g(i)], dstb[p], sem_d[p])

        def start_gather(i, p):
            pltpu.make_async_copy(eidx_hbm.at[0, g(i)],
                                  srcb[p], sem_s[p]).wait()
            pltpu.async_copy(tab_sh.at[srcb[p]], rows[p], sem_g[p])

        for k in range(NBUF):
            load_idx(k, k)
        for k in range(NBUF - 1):
            start_gather(k, k)

        def stage(i, p):
            qg = (p + NBUF - 1) % NBUF

            @pl.when(i < niter)
            def _():
                pltpu.make_async_copy(tab_sh.at[srcb[p]],
                                      rows[p], sem_g[p]).wait()
                pltpu.make_async_copy(eidx_hbm.at[1, g(i)],
                                      dstb[p], sem_d[p]).wait()
                pltpu.sync_copy(rows[p], acc_sh.at[dstb[p]], add=True)

            @pl.when(i + NBUF < niter)
            def _():
                load_idx(i + NBUF, p)

            @pl.when(i + NBUF - 1 < niter)
            def _():
                start_gather(i + NBUF - 1, qg)

        def body(j, carry):
            for p in range(NBUF):
                stage(NBUF * j + p, p)
            return carry

        nfull = niter_max // NBUF
        lax.fori_loop(0, nfull, body, 0)
        for i in range(NBUF * nfull, niter_max):
            stage(i, i % NBUF)
        plsc.subcore_barrier()

        _halfwise(c, lambda h: _rowwise(lambda r0, nr: pltpu.sync_copy(
            acc_sh.at[pl.ds(r0, nr)],
            out_hbm.at[pl.ds(r0, nr), pl.ds(h * DH, DH)]))(s))

    return sc_agg


@functools.lru_cache(maxsize=None)
def _make_sc_deg():
    NBUF = 4
    mesh = plsc.VectorSubcoreMesh(core_axis_name="c", subcore_axis_name="s",
                                  num_cores=NC, num_subcores=NS)
    niter_max = -(-NCHUNK // NW)
    full = NCHUNK - NW * (niter_max - 1)

    @functools.partial(
        pl.kernel,
        out_type=jax.ShapeDtypeStruct((N, 2 * DDEG), jnp.float32),
        mesh=mesh,
        scratch_types=[
            [pltpu.VMEM((C,), jnp.int32) for _ in range(NBUF)],
            pltpu.VMEM((C, DDEG), jnp.float32),
            pltpu.VMEM_SHARED((ACC_ROWS, DDEG), jnp.float32),
            [pltpu.SemaphoreType.DMA for _ in range(NBUF)],
        ],
        compiler_params=pltpu.CompilerParams(use_tc_tiling_on_sc=False),
    )
    def sc_deg(eidx_hbm, ones_hbm, zeros_hbm, out_hbm,
               dstb, ones_v, acc_sh, sem_d):
        c = lax.axis_index("c")
        s = lax.axis_index("s")
        wid = s * NC + c
        niter = jnp.where(wid < full, niter_max, niter_max - 1)

        pltpu.sync_copy(ones_hbm, ones_v)
        pltpu.sync_copy(zeros_hbm, acc_sh.at[pl.ds(s * ZROWS, ZROWS)])
        plsc.subcore_barrier()

        def g(i):
            return NW * i + wid

        for k in range(NBUF):
            pltpu.async_copy(eidx_hbm.at[1, g(k)], dstb[k], sem_d[k])

        def stage(i, p):
            @pl.when(i < niter)
            def _():
                pltpu.make_async_copy(eidx_hbm.at[1, g(i)],
                                      dstb[p], sem_d[p]).wait()
                pltpu.sync_copy(ones_v, acc_sh.at[dstb[p]], add=True)

            @pl.when(i + NBUF < niter)
            def _():
                pltpu.async_copy(eidx_hbm.at[1, g(i + NBUF)],
                                 dstb[p], sem_d[p])

        def body(j, carry):
            for p in range(NBUF):
                stage(NBUF * j + p, p)
            return carry

        nfull = niter_max // NBUF
        lax.fori_loop(0, nfull, body, 0)
        for i in range(NBUF * nfull, niter_max):
            stage(i, i % NBUF)
        plsc.subcore_barrier()

        _halfwise(c, lambda h: _rowwise(lambda r0, nr: pltpu.sync_copy(
            acc_sh.at[pl.ds(r0, nr)],
            out_hbm.at[pl.ds(r0, nr), pl.ds(h * DDEG, DDEG)]))(s))

    return sc_deg


def _dinv_of(g_ref):
    deg = g_ref[:, 0:1] + g_ref[:, DDEG:DDEG + 1]
    return 1.0 / jnp.maximum(deg, 1.0)


def _tc1_body(x_ref, w_ref, out_ref):
    out_ref[...] = jnp.dot(x_ref[...], w_ref[...],
                           preferred_element_type=jnp.float32)


def _tc1(x, w0):
    return pl.pallas_call(
        _tc1_body,
        grid=(GRID,),
        in_specs=[
            pl.BlockSpec((BM, F), lambda m: (m, 0)),
            pl.BlockSpec((F, F), lambda m: (0, 0)),
        ],
        out_specs=pl.BlockSpec((BM, F), lambda m: (m, 0)),
        out_shape=jax.ShapeDtypeStruct((N, F), jnp.float32),
    )(x, w0)


def _tc23_body(a_ref, g_ref, w_ref, b_ref, out_ref):
    dinv = _dinv_of(g_ref)
    h = jnp.maximum(a_ref[...] * dinv + b_ref[...], 0.0)
    out_ref[...] = jnp.dot(h, w_ref[...], preferred_element_type=jnp.float32)


def _tc23(a, gdeg, w, b, DO):
    return pl.pallas_call(
        _tc23_body,
        grid=(GRID,),
        in_specs=[
            pl.BlockSpec((BM, F), lambda m: (m, 0)),
            pl.BlockSpec((BM, 2 * DDEG), lambda m: (m, 0)),
            pl.BlockSpec((F, DO), lambda m: (0, 0)),
            pl.BlockSpec((1, F), lambda m: (0, 0)),
        ],
        out_specs=pl.BlockSpec((BM, DO), lambda m: (m, 0)),
        out_shape=jax.ShapeDtypeStruct((N, DO), jnp.float32),
    )(a, gdeg, w, b)


def _tc4_body(a_ref, g_ref, b_ref, out_ref):
    dinv = _dinv_of(g_ref)
    out_ref[...] = a_ref[:, :NCLS] * dinv + b_ref[...]


def _tc4(a, gdeg, b2):
    return pl.pallas_call(
        _tc4_body,
        grid=(GRID,),
        in_specs=[
            pl.BlockSpec((BM, D2), lambda m: (m, 0)),
            pl.BlockSpec((BM, 2 * DDEG), lambda m: (m, 0)),
            pl.BlockSpec((1, NCLS), lambda m: (0, 0)),
        ],
        out_specs=pl.BlockSpec((BM, NCLS), lambda m: (m, 0)),
        out_shape=jax.ShapeDtypeStruct((N, NCLS), jnp.float32),
    )(a, gdeg, b2)


def kernel(features, edge_index, W0, b0, W1, b1, W2, b2):
    eidx = edge_index.reshape(2, NCHUNK, C)
    w2p = jnp.pad(W2, ((0, 0), (0, D2 - NCLS)))
    zeros_h = jnp.zeros((ZROWS, FH), jnp.float32)

    gdeg = _make_sc_deg()(eidx, jnp.ones((C, DDEG), jnp.float32),
                          jnp.zeros((ZROWS, DDEG), jnp.float32))

    hw0 = _tc1(features, W0)
    a0 = _make_sc_agg(FH, 4)(hw0, eidx, zeros_h, gdeg)
    hw1 = _tc23(a0, gdeg, W1, b0[None, :], F)
    a1 = _make_sc_agg(FH, 4)(hw1, eidx, zeros_h, gdeg)
    hw2 = _tc23(a1, gdeg, w2p, b1[None, :], D2)
    a2 = _make_sc_agg(D2H, 4)(hw2, eidx,
                              jnp.zeros((ZROWS, D2H), jnp.float32), gdeg)
    return _tc4(a2, gdeg, b2[None, :])


# speedup vs baseline: 1.0161x; 1.0161x over previous
import functools

import jax
import jax.numpy as jnp
from jax import lax
from jax.experimental import pallas as pl
from jax.experimental.pallas import tpu as pltpu
from jax.experimental.pallas import tpu_sc as plsc

N = 10000
E = 320000
F = 128
FH = 64
D2 = 48
D2H = 24
DDEG = 16
NCLS = 40

NC, NS = 2, 16
NW = NC * NS
C = 128
NCHUNK = E // C
ZROWS = 632
ACC_ROWS = NS * ZROWS
ROW_TAIL = N - (NS - 1) * ZROWS

BM = 2000
GRID = N // BM


def _rowwise(fn):
    def run(s):
        @pl.when(s < NS - 1)
        def _():
            fn(s * ZROWS, ZROWS)

        @pl.when(s == NS - 1)
        def _():
            fn((NS - 1) * ZROWS, ROW_TAIL)
    return run


def _halfwise(c, fn):
    @pl.when(c == 0)
    def _():
        fn(0)

    @pl.when(c == 1)
    def _():
        fn(1)


@functools.lru_cache(maxsize=None)
def _make_sc_agg(DH, NBUF):
    mesh = plsc.VectorSubcoreMesh(core_axis_name="c", subcore_axis_name="s",
                                  num_cores=NC, num_subcores=NS)
    niter_max = -(-NCHUNK // NS)
    full = NCHUNK - NS * (niter_max - 1)

    @functools.partial(
        pl.kernel,
        out_type=jax.ShapeDtypeStruct((N, 2 * DH), jnp.float32),
        mesh=mesh,
        scratch_types=[
            [pltpu.VMEM((C,), jnp.int32) for _ in range(NBUF)],
            [pltpu.VMEM((C,), jnp.int32) for _ in range(NBUF)],
            [pltpu.VMEM((C, DH), jnp.float32) for _ in range(NBUF)],
            pltpu.VMEM_SHARED((N, DH), jnp.float32),
            pltpu.VMEM_SHARED((ACC_ROWS, DH), jnp.float32),
            [pltpu.SemaphoreType.DMA for _ in range(NBUF)],
            [pltpu.SemaphoreType.DMA for _ in range(NBUF)],
            [pltpu.SemaphoreType.DMA for _ in range(NBUF)],
        ],
        compiler_params=pltpu.CompilerParams(use_tc_tiling_on_sc=False),
    )
    def sc_agg(hw_hbm, eidx_hbm, zeros_hbm, dep_hbm, out_hbm,
               srcb, dstb, rows, tab_sh, acc_sh, sem_s, sem_d, sem_g):
        del dep_hbm
        c = lax.axis_index("c")
        s = lax.axis_index("s")
        niter = jnp.where(s < full, niter_max, niter_max - 1)

        _halfwise(c, lambda h: _rowwise(lambda r0, nr: pltpu.sync_copy(
            hw_hbm.at[pl.ds(r0, nr), pl.ds(h * DH, DH)],
            tab_sh.at[pl.ds(r0, nr)]))(s))
        pltpu.sync_copy(zeros_hbm, acc_sh.at[pl.ds(s * ZROWS, ZROWS)])
        plsc.subcore_barrier()

        def g(i):
            return NS * i + s

        def load_idx(i, p):
            pltpu.async_copy(eidx_hbm.at[0, g(i)], srcb[p], sem_s[p])
            pltpu.async_copy(eidx_hbm.at[1, g(i)], dstb[p], sem_d[p])

        def start_gather(i, p):
            pltpu.make_async_copy(eidx_hbm.at[0, g(i)],
                                  srcb[p], sem_s[p]).wait()
            pltpu.async_copy(tab_sh.at[srcb[p]], rows[p], sem_g[p])

        for k in range(NBUF):
            load_idx(k, k)
        for k in range(NBUF - 1):
            start_gather(k, k)

        def stage(i, p):
            qg = (p + NBUF - 1) % NBUF

            @pl.when(i < niter)
            def _():
                pltpu.make_async_copy(tab_sh.at[srcb[p]],
                                      rows[p], sem_g[p]).wait()
                pltpu.make_async_copy(eidx_hbm.at[1, g(i)],
                                      dstb[p], sem_d[p]).wait()
                pltpu.sync_copy(rows[p], acc_sh.at[dstb[p]], add=True)

            @pl.when(i + NBUF < niter)
            def _():
                load_idx(i + NBUF, p)

            @pl.when(i + NBUF - 1 < niter)
            def _():
                start_gather(i + NBUF - 1, qg)

        def body(j, carry):
            for p in range(NBUF):
                stage(NBUF * j + p, p)
            return carry

        nfull = niter_max // NBUF
        lax.fori_loop(0, nfull, body, 0)
        for i in range(NBUF * nfull, niter_max):
            stage(i, i % NBUF)
        plsc.subcore_barrier()

        _halfwise(c, lambda h: _rowwise(lambda r0, nr: pltpu.sync_copy(
            acc_sh.at[pl.ds(r0, nr)],
            out_hbm.at[pl.ds(r0, nr), pl.ds(h * DH, DH)]))(s))

    return sc_agg


@functools.lru_cache(maxsize=None)
def _make_sc_deg():
    NBUF = 4
    mesh = plsc.VectorSubcoreMesh(core_axis_name="c", subcore_axis_name="s",
                                  num_cores=NC, num_subcores=NS)
    niter_max = -(-NCHUNK // NW)
    full = NCHUNK - NW * (niter_max - 1)

    @functools.partial(
        pl.kernel,
        out_type=jax.ShapeDtypeStruct((N, 2 * DDEG), jnp.float32),
        mesh=mesh,
        scratch_types=[
            [pltpu.VMEM((C,), jnp.int32) for _ in range(NBUF)],
            pltpu.VMEM((C, DDEG), jnp.float32),
            pltpu.VMEM_SHARED((ACC_ROWS, DDEG), jnp.float32),
            [pltpu.SemaphoreType.DMA for _ in range(NBUF)],
        ],
        compiler_params=pltpu.CompilerParams(use_tc_tiling_on_sc=False),
    )
    def sc_deg(eidx_hbm, ones_hbm, zeros_hbm, out_hbm,
               dstb, ones_v, acc_sh, sem_d):
        c = lax.axis_index("c")
        s = lax.axis_index("s")
        wid = s * NC + c
        niter = jnp.where(wid < full, niter_max, niter_max - 1)

        pltpu.sync_copy(ones_hbm, ones_v)
        pltpu.sync_copy(zeros_hbm, acc_sh.at[pl.ds(s * ZROWS, ZROWS)])
        plsc.subcore_barrier()

        def g(i):
            return NW * i + wid

        for k in range(NBUF):
            pltpu.async_copy(eidx_hbm.at[1, g(k)], dstb[k], sem_d[k])

        def stage(i, p):
            @pl.when(i < niter)
            def _():
                pltpu.make_async_copy(eidx_hbm.at[1, g(i)],
                                      dstb[p], sem_d[p]).wait()
                pltpu.sync_copy(ones_v, acc_sh.at[dstb[p]], add=True)

            @pl.when(i + NBUF < niter)
            def _():
                pltpu.async_copy(eidx_hbm.at[1, g(i + NBUF)],
                                 dstb[p], sem_d[p])

        def body(j, carry):
            for p in range(NBUF):
                stage(NBUF * j + p, p)
            return carry

        nfull = niter_max // NBUF
        lax.fori_loop(0, nfull, body, 0)
        for i in range(NBUF * nfull, niter_max):
            stage(i, i % NBUF)
        plsc.subcore_barrier()

        _halfwise(c, lambda h: _rowwise(lambda r0, nr: pltpu.sync_copy(
            acc_sh.at[pl.ds(r0, nr)],
            out_hbm.at[pl.ds(r0, nr), pl.ds(h * DDEG, DDEG)]))(s))

    return sc_deg


def _dinv_of(g_ref):
    deg = g_ref[:, 0:1] + g_ref[:, DDEG:DDEG + 1]
    return 1.0 / jnp.maximum(deg, 1.0)


def _tc1_body(x_ref, w_ref, out_ref):
    out_ref[...] = jnp.dot(x_ref[...], w_ref[...],
                           preferred_element_type=jnp.float32)


def _tc1(x, w0):
    return pl.pallas_call(
        _tc1_body,
        grid=(GRID,),
        in_specs=[
            pl.BlockSpec((BM, F), lambda m: (m, 0)),
            pl.BlockSpec((F, F), lambda m: (0, 0)),
        ],
        out_specs=pl.BlockSpec((BM, F), lambda m: (m, 0)),
        out_shape=jax.ShapeDtypeStruct((N, F), jnp.float32),
    )(x, w0)


def _tc23_body(a_ref, g_ref, w_ref, b_ref, out_ref):
    dinv = _dinv_of(g_ref)
    h = jnp.maximum(a_ref[...] * dinv + b_ref[...], 0.0)
    out_ref[...] = jnp.dot(h, w_ref[...], preferred_element_type=jnp.float32)


def _tc23(a, gdeg, w, b, DO):
    return pl.pallas_call(
        _tc23_body,
        grid=(GRID,),
        in_specs=[
            pl.BlockSpec((BM, F), lambda m: (m, 0)),
            pl.BlockSpec((BM, 2 * DDEG), lambda m: (m, 0)),
            pl.BlockSpec((F, DO), lambda m: (0, 0)),
            pl.BlockSpec((1, F), lambda m: (0, 0)),
        ],
        out_specs=pl.BlockSpec((BM, DO), lambda m: (m, 0)),
        out_shape=jax.ShapeDtypeStruct((N, DO), jnp.float32),
    )(a, gdeg, w, b)


def _tc4_body(a_ref, g_ref, b_ref, out_ref):
    dinv = _dinv_of(g_ref)
    out_ref[...] = a_ref[:, :NCLS] * dinv + b_ref[...]


def _tc4(a, gdeg, b2):
    return pl.pallas_call(
        _tc4_body,
        grid=(GRID,),
        in_specs=[
            pl.BlockSpec((BM, D2), lambda m: (m, 0)),
            pl.BlockSpec((BM, 2 * DDEG), lambda m: (m, 0)),
            pl.BlockSpec((1, NCLS), lambda m: (0, 0)),
        ],
        out_specs=pl.BlockSpec((BM, NCLS), lambda m: (m, 0)),
        out_shape=jax.ShapeDtypeStruct((N, NCLS), jnp.float32),
    )(a, gdeg, b2)


def kernel(features, edge_index, W0, b0, W1, b1, W2, b2):
    eidx = edge_index.reshape(2, NCHUNK, C)
    w2p = jnp.pad(W2, ((0, 0), (0, D2 - NCLS)))
    zeros_h = jnp.zeros((ZROWS, FH), jnp.float32)

    gdeg = _make_sc_deg()(eidx, jnp.ones((C, DDEG), jnp.float32),
                          jnp.zeros((ZROWS, DDEG), jnp.float32))

    hw0 = _tc1(features, W0)
    a0 = _make_sc_agg(FH, 4)(hw0, eidx, zeros_h, gdeg)
    hw1 = _tc23(a0, gdeg, W1, b0[None, :], F)
    a1 = _make_sc_agg(FH, 4)(hw1, eidx, zeros_h, gdeg)
    hw2 = _tc23(a1, gdeg, w2p, b1[None, :], D2)
    a2 = _make_sc_agg(D2H, 4)(hw2, eidx,
                              jnp.zeros((ZROWS, D2H), jnp.float32), gdeg)
    return _tc4(a2, gdeg, b2[None, :])
